# Initial kernel scaffold; baseline (speedup 1.0000x reference)
import jax
import jax.numpy as jnp
from jax.experimental import pallas as pl


def kernel(x, y, W0, W1, W2, W3, W4, Wy1, by1, Wy2, by2, L0, L1, L2, bL2):
    raise NotImplementedError("write your pallas kernel here")


# speedup vs baseline: 9.2186x; 9.2186x over previous
import functools

import jax
import jax.numpy as jnp
from jax import lax
from jax.experimental import pallas as pl
from jax.experimental.pallas import tpu as pltpu
from jax.experimental.pallas import tpu_sc as plsc

B = 8
N = 2048
K = 20
BN = B * N
NT = 256
NP = 64
NT2 = 512
NW = 32
CHUNK = BN // NW
TW = 128


def _leaky(v):
    return jnp.where(v > 0, v, 0.2 * v)


@functools.lru_cache(maxsize=None)
def _knn_tc(C, FW):
    nt_blocks = N // NT

    def body(f_ref, ft_ref, idx_ref):
        b = pl.program_id(0)
        ft = f_ref[...][:, :C]
        fbt = ft_ref[0]
        inner = -2.0 * lax.dot_general(ft, fbt, (((1,), (0,)), ((), ())),
                                       preferred_element_type=jnp.float32)
        xx_row = jnp.sum(fbt * fbt, axis=0, keepdims=True)
        xx_col = jnp.sum(ft * ft, axis=1, keepdims=True)
        s = ((-xx_col) - inner) - xx_row
        iota = lax.broadcasted_iota(jnp.int32, (NT, N), 1)
        cols = []
        for _ in range(K):
            m = jnp.max(s, axis=1, keepdims=True)
            idxv = jnp.min(jnp.where(s == m, iota, N), axis=1, keepdims=True)
            cols.append(idxv)
            s = jnp.where(iota == idxv, -jnp.inf, s)
        idx_ref[...] = jnp.concatenate(cols, axis=1) + b * N

    return pl.pallas_call(
        body,
        grid=(B, nt_blocks),
        in_specs=[
            pl.BlockSpec((NT, FW), lambda b, t: (b * nt_blocks + t, 0)),
            pl.BlockSpec((1, C, N), lambda b, t: (b, 0, 0)),
        ],
        out_specs=pl.BlockSpec((NT, K), lambda b, t: (b * nt_blocks + t, 0)),
        out_shape=jax.ShapeDtypeStruct((BN, K), jnp.int32),
    )


@functools.lru_cache(maxsize=None)
def _edge_gather(EW, SS, GP):
    HC = EW // 2
    NG = SS // GP
    NSUP = CHUNK // SS
    mesh = plsc.VectorSubcoreMesh(core_axis_name="c", subcore_axis_name="s")

    @functools.partial(
        pl.kernel,
        out_type=jax.ShapeDtypeStruct((BN * K, EW), jnp.float32),
        mesh=mesh,
        scratch_types=[
            pltpu.VMEM((CHUNK * K,), jnp.int32),
            pltpu.VMEM((SS * K, TW), jnp.float32),
            pltpu.VMEM((SS, TW), jnp.float32),
            pltpu.VMEM((SS * K, EW), jnp.float32),
            pltpu.SemaphoreType.DMA,
        ],
    )
    def eg(t_hbm, idx_hbm, e_hbm, idx_v, rows_v, fi_v, ev, sem):
        wid = lax.axis_index("s") * 2 + lax.axis_index("c")
        base = pl.multiple_of(wid * CHUNK, CHUNK)
        pltpu.sync_copy(idx_hbm.at[pl.ds(base * K, CHUNK * K)], idx_v)

        def sup_body(g, carry):
            p0 = pl.multiple_of(g * SS, SS)
            handles = []
            for d in range(NG):
                isl = idx_v.at[pl.ds(pl.multiple_of((p0 + d * GP) * K, GP * K),
                                     GP * K)]
                dst = rows_v.at[pl.ds(d * GP * K, GP * K)]
                handles.append(pltpu.async_copy(t_hbm.at[isl], dst, sem))
            pltpu.sync_copy(t_hbm.at[pl.ds(base + p0, SS)], fi_v)
            for h in handles:
                h.wait()

            def pt_body(p, c2):
                r0 = p * K
                for j in range(HC // 16):
                    sl = pl.ds(j * 16, 16)
                    sl2 = pl.ds(HC + j * 16, 16)
                    fij = fi_v[p, sl]
                    for r in range(K):
                        ev[r0 + r, sl] = rows_v[r0 + r, sl] - fij
                        ev[r0 + r, sl2] = fij
                return c2

            lax.fori_loop(0, SS, pt_body, 0)
            pltpu.sync_copy(ev, e_hbm.at[pl.ds((base + p0) * K, SS * K)])
            return carry

        lax.fori_loop(0, NSUP, sup_body, 0)

    return eg


def _edge_features(table, idx_flat, EW):
    SS = 8 if EW == 256 else 16
    return _edge_gather(EW, SS, 4)(table, idx_flat)


@functools.lru_cache(maxsize=None)
def _edge_mm(EW, O, OPAD):
    blocks = BN // NP

    def body(e_ref, w_ref, out_ref):
        h = lax.dot_general(e_ref[...], w_ref[...], (((1,), (0,)), ((), ())),
                            preferred_element_type=jnp.float32)
        h = _leaky(h)
        m = jnp.max(h.reshape(NP, K, O), axis=1)
        if OPAD != O:
            m = jnp.concatenate(
                [m, jnp.zeros((NP, OPAD - O), jnp.float32)], axis=1)
        out_ref[...] = m

    return pl.pallas_call(
        body,
        grid=(blocks,),
        in_specs=[
            pl.BlockSpec((NP * K, EW), lambda t: (t, 0)),
            pl.BlockSpec((EW, O), lambda t: (0, 0)),
        ],
        out_specs=pl.BlockSpec((NP, OPAD), lambda t: (t, 0)),
        out_shape=jax.ShapeDtypeStruct((BN, OPAD), jnp.float32),
    )


@functools.lru_cache(maxsize=None)
def _pool_tc():
    nt_blocks = N // NT2

    def body(f1_ref, f2_ref, f3_ref, f4_ref, w_ref, out_ref):
        t = pl.program_id(1)
        cat = jnp.concatenate(
            [f1_ref[...][:, :64], f2_ref[...][:, :64],
             f3_ref[...], f4_ref[...]], axis=1)
        h = lax.dot_general(cat, w_ref[...], (((1,), (0,)), ((), ())),
                            preferred_element_type=jnp.float32)
        part = jnp.max(_leaky(h), axis=0, keepdims=True)

        @pl.when(t == 0)
        def _init():
            out_ref[0] = part

        @pl.when(t != 0)
        def _acc():
            out_ref[0] = jnp.maximum(out_ref[0], part)

    return pl.pallas_call(
        body,
        grid=(B, nt_blocks),
        in_specs=[
            pl.BlockSpec((NT2, 128), lambda b, t: (b * nt_blocks + t, 0)),
            pl.BlockSpec((NT2, 128), lambda b, t: (b * nt_blocks + t, 0)),
            pl.BlockSpec((NT2, 128), lambda b, t: (b * nt_blocks + t, 0)),
            pl.BlockSpec((NT2, 256), lambda b, t: (b * nt_blocks + t, 0)),
            pl.BlockSpec((512, 1024), lambda b, t: (0, 0)),
        ],
        out_specs=pl.BlockSpec((1, 1, 1024), lambda b, t: (b, 0, 0)),
        out_shape=jax.ShapeDtypeStruct((B, 1, 1024), jnp.float32),
    )


@functools.lru_cache(maxsize=None)
def _head_tc():
    def body(nx_ref, y_ref, wy1_ref, by1_ref, wy2_ref, by2_ref,
             l0_ref, l1_ref, l2_ref, bl2_ref, out_ref):
        dn = (((1,), (0,)), ((), ()))
        yy = _leaky(lax.dot_general(y_ref[...], wy1_ref[...], dn,
                                    preferred_element_type=jnp.float32)
                    + by1_ref[...])
        yy = _leaky(lax.dot_general(yy, wy2_ref[...], dn,
                                    preferred_element_type=jnp.float32)
                    + by2_ref[...])
        z = jnp.concatenate([nx_ref[...], yy], axis=1)
        z = _leaky(lax.dot_general(z, l0_ref[...], dn,
                                   preferred_element_type=jnp.float32))
        z = _leaky(lax.dot_general(z, l1_ref[...], dn,
                                   preferred_element_type=jnp.float32))
        out_ref[...] = lax.dot_general(z, l2_ref[...], dn,
                                       preferred_element_type=jnp.float32) \
            + bl2_ref[...]

    return pl.pallas_call(
        body,
        out_shape=jax.ShapeDtypeStruct((B, 1), jnp.float32),
    )


def kernel(x, y, W0, W1, W2, W3, W4, Wy1, by1, Wy2, by2, L0, L1, L2, bL2):
    f = x.reshape(BN, 6)
    table = jnp.pad(f, ((0, 0), (0, TW - 6)))
    ft = jnp.transpose(x, (0, 2, 1))

    feats = []
    for W, C, O in ((W0, 6, 64), (W1, 64, 64), (W2, 64, 128), (W3, 128, 256)):
        FW = TW if C != 6 else 6
        idx = _knn_tc(C, FW)(f if C == 6 else feats[-1], ft)
        HC = max(16, C)
        EW = 2 * HC
        e = _edge_features(table, idx.reshape(BN * K), EW)
        wcat = jnp.zeros((EW, O), jnp.float32)
        wcat = wcat.at[:C].set(W[:, :C].T).at[HC:HC + C].set(W[:, C:].T)
        OPAD = max(O, TW)
        fpad = _edge_mm(EW, O, OPAD)(e, wcat)
        feats.append(fpad)
        if O != 256:
            table = fpad
            ft = jnp.transpose(fpad[:, :O].reshape(B, N, O), (0, 2, 1))

    w4t = W4.T
    nx = _pool_tc()(feats[0], feats[1], feats[2], feats[3], w4t)
    nx = nx.reshape(B, 1024)

    z = _head_tc()(nx, y, Wy1.T, by1.reshape(1, 16), Wy2.T,
                   by2.reshape(1, 64), L0.T, L1.T, L2.T, bL2.reshape(1, 1))
    return z


# speedup vs baseline: 10.8832x; 1.1806x over previous
import functools

import jax
import jax.numpy as jnp
from jax import lax
from jax.experimental import pallas as pl
from jax.experimental.pallas import tpu as pltpu
from jax.experimental.pallas import tpu_sc as plsc

B = 8
N = 2048
K = 20
NB = 4
NBN = NB * N
NT = 256
NP = 64
NT2 = 512
NW = 32
CHUNK = NBN // NW
TW = 128


def _leaky(v):
    return jnp.where(v > 0, v, 0.2 * v)


@functools.lru_cache(maxsize=None)
def _knn_tc(C, FW):
    nt_blocks = N // NT

    def body(f_ref, ft_ref, idx_ref):
        b = pl.program_id(0)
        ft = f_ref[...][:, :C]
        fbt = ft_ref[0]
        inner = -2.0 * lax.dot_general(ft, fbt, (((1,), (0,)), ((), ())),
                                       preferred_element_type=jnp.float32)
        xx_row = jnp.sum(fbt * fbt, axis=0, keepdims=True)
        xx_col = jnp.sum(ft * ft, axis=1, keepdims=True)
        s = ((-xx_col) - inner) - xx_row
        iota = lax.broadcasted_iota(jnp.int32, (NT, N), 1)
        cols = []
        m = jnp.max(s, axis=1, keepdims=True)
        for t in range(K):
            idxv = jnp.min(jnp.where(s == m, iota, N), axis=1, keepdims=True)
            cols.append(idxv)
            if t < K - 1:
                s = jnp.where(iota == idxv, -jnp.inf, s)
                m = jnp.max(s, axis=1, keepdims=True)
        idx_ref[...] = jnp.concatenate(cols, axis=1) + b * N

    return pl.pallas_call(
        body,
        grid=(NB, nt_blocks),
        in_specs=[
            pl.BlockSpec((NT, FW), lambda b, t: (b * nt_blocks + t, 0)),
            pl.BlockSpec((1, C, N), lambda b, t: (b, 0, 0)),
        ],
        out_specs=pl.BlockSpec((NT, K), lambda b, t: (b * nt_blocks + t, 0)),
        out_shape=jax.ShapeDtypeStruct((NBN, K), jnp.int32),
    )


@functools.lru_cache(maxsize=None)
def _edge_gather(EW, SS, GP):
    HC = EW // 2
    NG = SS // GP
    NSUP = CHUNK // SS
    mesh = plsc.VectorSubcoreMesh(core_axis_name="c", subcore_axis_name="s")

    @functools.partial(
        pl.kernel,
        out_type=jax.ShapeDtypeStruct((NBN * K, EW), jnp.float32),
        mesh=mesh,
        scratch_types=[
            pltpu.VMEM((CHUNK * K,), jnp.int32),
            pltpu.VMEM((SS * K, TW), jnp.float32),
            pltpu.VMEM((SS, TW), jnp.float32),
            pltpu.VMEM((SS * K, EW), jnp.float32),
            pltpu.SemaphoreType.DMA,
        ],
    )
    def eg(t_hbm, idx_hbm, e_hbm, idx_v, rows_v, fi_v, ev, sem):
        wid = lax.axis_index("s") * 2 + lax.axis_index("c")
        base = pl.multiple_of(wid * CHUNK, CHUNK)
        pltpu.sync_copy(idx_hbm.at[pl.ds(base * K, CHUNK * K)], idx_v)

        def sup_body(g, carry):
            p0 = pl.multiple_of(g * SS, SS)
            handles = []
            for d in range(NG):
                isl = idx_v.at[pl.ds(pl.multiple_of((p0 + d * GP) * K, GP * K),
                                     GP * K)]
                dst = rows_v.at[pl.ds(d * GP * K, GP * K)]
                handles.append(pltpu.async_copy(t_hbm.at[isl], dst, sem))
            pltpu.sync_copy(t_hbm.at[pl.ds(base + p0, SS)], fi_v)
            for h in handles:
                h.wait()

            def pt_body(p, c2):
                r0 = p * K
                for j in range(HC // 16):
                    sl = pl.ds(j * 16, 16)
                    sl2 = pl.ds(HC + j * 16, 16)
                    fij = fi_v[p, sl]
                    for r in range(K):
                        ev[r0 + r, sl] = rows_v[r0 + r, sl] - fij
                        ev[r0 + r, sl2] = fij
                return c2

            lax.fori_loop(0, SS, pt_body, 0)
            pltpu.sync_copy(ev, e_hbm.at[pl.ds((base + p0) * K, SS * K)])
            return carry

        lax.fori_loop(0, NSUP, sup_body, 0)

    return eg


def _edge_features(table, idx_flat, EW):
    SS = 8 if EW == 256 else 16
    return _edge_gather(EW, SS, 4)(table, idx_flat)


@functools.lru_cache(maxsize=None)
def _edge_mm(EW, O, OPAD):
    blocks = NBN // NP

    def body(e_ref, w_ref, out_ref):
        h = lax.dot_general(e_ref[...], w_ref[...], (((1,), (0,)), ((), ())),
                            preferred_element_type=jnp.float32)
        h = _leaky(h)
        m = jnp.max(h.reshape(NP, K, O), axis=1)
        if OPAD != O:
            m = jnp.concatenate(
                [m, jnp.zeros((NP, OPAD - O), jnp.float32)], axis=1)
        out_ref[...] = m

    return pl.pallas_call(
        body,
        grid=(blocks,),
        in_specs=[
            pl.BlockSpec((NP * K, EW), lambda t: (t, 0)),
            pl.BlockSpec((EW, O), lambda t: (0, 0)),
        ],
        out_specs=pl.BlockSpec((NP, OPAD), lambda t: (t, 0)),
        out_shape=jax.ShapeDtypeStruct((NBN, OPAD), jnp.float32),
    )


@functools.lru_cache(maxsize=None)
def _pool_tc():
    nt_blocks = N // NT2

    def body(f1_ref, f2_ref, f3_ref, f4_ref, w_ref, out_ref):
        t = pl.program_id(1)
        cat = jnp.concatenate(
            [f1_ref[...][:, :64], f2_ref[...][:, :64],
             f3_ref[...], f4_ref[...]], axis=1)
        h = lax.dot_general(cat, w_ref[...], (((1,), (0,)), ((), ())),
                            preferred_element_type=jnp.float32)
        part = jnp.max(_leaky(h), axis=0, keepdims=True)

        @pl.when(t == 0)
        def _init():
            out_ref[0] = part

        @pl.when(t != 0)
        def _acc():
            out_ref[0] = jnp.maximum(out_ref[0], part)

    return pl.pallas_call(
        body,
        grid=(NB, nt_blocks),
        in_specs=[
            pl.BlockSpec((NT2, 128), lambda b, t: (b * nt_blocks + t, 0)),
            pl.BlockSpec((NT2, 128), lambda b, t: (b * nt_blocks + t, 0)),
            pl.BlockSpec((NT2, 128), lambda b, t: (b * nt_blocks + t, 0)),
            pl.BlockSpec((NT2, 256), lambda b, t: (b * nt_blocks + t, 0)),
            pl.BlockSpec((512, 1024), lambda b, t: (0, 0)),
        ],
        out_specs=pl.BlockSpec((1, 1, 1024), lambda b, t: (b, 0, 0)),
        out_shape=jax.ShapeDtypeStruct((NB, 1, 1024), jnp.float32),
    )


@functools.lru_cache(maxsize=None)
def _head_tc():
    def body(nx_ref, y_ref, wy1_ref, by1_ref, wy2_ref, by2_ref,
             l0_ref, l1_ref, l2_ref, bl2_ref, out_ref):
        dn = (((1,), (0,)), ((), ()))
        yy = _leaky(lax.dot_general(y_ref[...], wy1_ref[...], dn,
                                    preferred_element_type=jnp.float32)
                    + by1_ref[...])
        yy = _leaky(lax.dot_general(yy, wy2_ref[...], dn,
                                    preferred_element_type=jnp.float32)
                    + by2_ref[...])
        z = jnp.concatenate([nx_ref[...], yy], axis=1)
        z = _leaky(lax.dot_general(z, l0_ref[...], dn,
                                   preferred_element_type=jnp.float32))
        z = _leaky(lax.dot_general(z, l1_ref[...], dn,
                                   preferred_element_type=jnp.float32))
        out_ref[...] = lax.dot_general(z, l2_ref[...], dn,
                                       preferred_element_type=jnp.float32) \
            + bl2_ref[...]

    return pl.pallas_call(
        body,
        out_shape=jax.ShapeDtypeStruct((B, 1), jnp.float32),
    )


def _chain(xc, wcats, w4t):
    f = xc.reshape(NBN, 6)
    table = jnp.pad(f, ((0, 0), (0, TW - 6)))
    ft = jnp.transpose(xc, (0, 2, 1))

    feats = []
    for (wcat, C, O) in wcats:
        FW = TW if C != 6 else 6
        idx = _knn_tc(C, FW)(f if C == 6 else feats[-1], ft)
        HC = max(16, C)
        EW = 2 * HC
        e = _edge_features(table, idx.reshape(NBN * K), EW)
        OPAD = max(O, TW)
        fpad = _edge_mm(EW, O, OPAD)(e, wcat)
        feats.append(fpad)
        if O != 256:
            table = fpad
            ft = jnp.transpose(fpad[:, :O].reshape(NB, N, O), (0, 2, 1))

    return _pool_tc()(feats[0], feats[1], feats[2], feats[3], w4t)


def kernel(x, y, W0, W1, W2, W3, W4, Wy1, by1, Wy2, by2, L0, L1, L2, bL2):
    wcats = []
    for W, C, O in ((W0, 6, 64), (W1, 64, 64), (W2, 64, 128), (W3, 128, 256)):
        HC = max(16, C)
        wcat = jnp.zeros((2 * HC, O), jnp.float32)
        wcat = wcat.at[:C].set(W[:, :C].T).at[HC:HC + C].set(W[:, C:].T)
        wcats.append((wcat, C, O))
    w4t = W4.T

    nxa = _chain(x[:NB], wcats, w4t)
    nxb = _chain(x[NB:], wcats, w4t)
    nx = jnp.concatenate([nxa, nxb], axis=0).reshape(B, 1024)

    z = _head_tc()(nx, y, Wy1.T, by1.reshape(1, 16), Wy2.T,
                   by2.reshape(1, 64), L0.T, L1.T, L2.T, bL2.reshape(1, 1))
    return z


# speedup vs baseline: 14.3024x; 1.3142x over previous
import functools

import jax
import jax.numpy as jnp
from jax import lax
from jax.experimental import pallas as pl
from jax.experimental.pallas import tpu as pltpu
from jax.experimental.pallas import tpu_sc as plsc

B = 8
N = 2048
K = 20
NB = 4
NBN = NB * N
NT = 256
NP = 64
NT2 = 512
NW = 32
CHUNK = NBN // NW
TW = 128


def _leaky(v):
    return jnp.where(v > 0, v, 0.2 * v)


@functools.lru_cache(maxsize=None)
def _knn_tc(C, FW):
    nt_blocks = N // NT

    def body(f_ref, ft_ref, idx_ref):
        b = pl.program_id(0)
        ft = f_ref[...][:, :C]
        fbt = ft_ref[0]
        inner = -2.0 * lax.dot_general(ft, fbt, (((1,), (0,)), ((), ())),
                                       preferred_element_type=jnp.float32)
        xx_row = jnp.sum(fbt * fbt, axis=0, keepdims=True)
        xx_col = jnp.sum(ft * ft, axis=1, keepdims=True)
        s = ((-xx_col) - inner) - xx_row
        iota = lax.broadcasted_iota(jnp.int32, (NT, N), 1).astype(jnp.float32)
        cols = []
        m = jnp.max(s, axis=1, keepdims=True)
        for t in range(K):
            eqm = s == m
            idxv = jnp.min(jnp.where(eqm, iota, jnp.float32(N)),
                           axis=1, keepdims=True)
            cols.append(idxv.astype(jnp.int32))
            if t < K - 1:
                s = jnp.where(eqm, -jnp.inf, s)
                m = jnp.max(s, axis=1, keepdims=True)
        idx_ref[...] = jnp.concatenate(cols, axis=1) + b * N

    return pl.pallas_call(
        body,
        grid=(NB, nt_blocks),
        in_specs=[
            pl.BlockSpec((NT, FW), lambda b, t: (b * nt_blocks + t, 0)),
            pl.BlockSpec((1, C, N), lambda b, t: (b, 0, 0)),
        ],
        out_specs=pl.BlockSpec((NT, K), lambda b, t: (b * nt_blocks + t, 0)),
        out_shape=jax.ShapeDtypeStruct((NBN, K), jnp.int32),
    )


@functools.lru_cache(maxsize=None)
def _edge_gather(EW, SS, GP):
    HC = EW // 2
    NG = SS // GP
    NSUP = CHUNK // SS
    mesh = plsc.VectorSubcoreMesh(core_axis_name="c", subcore_axis_name="s")

    @functools.partial(
        pl.kernel,
        out_type=jax.ShapeDtypeStruct((NBN * K, EW), jnp.float32),
        mesh=mesh,
        scratch_types=[
            pltpu.VMEM((CHUNK * K,), jnp.int32),
            pltpu.VMEM((SS * K, TW), jnp.float32),
            pltpu.VMEM((SS, TW), jnp.float32),
            pltpu.VMEM((SS * K, EW), jnp.float32),
            pltpu.SemaphoreType.DMA,
        ],
    )
    def eg(t_hbm, idx_hbm, e_hbm, idx_v, rows_v, fi_v, ev, sem):
        wid = lax.axis_index("s") * 2 + lax.axis_index("c")
        base = pl.multiple_of(wid * CHUNK, CHUNK)
        pltpu.sync_copy(idx_hbm.at[pl.ds(base * K, CHUNK * K)], idx_v)

        def sup_body(g, carry):
            p0 = pl.multiple_of(g * SS, SS)
            handles = []
            for d in range(NG):
                isl = idx_v.at[pl.ds(pl.multiple_of((p0 + d * GP) * K, GP * K),
                                     GP * K)]
                dst = rows_v.at[pl.ds(d * GP * K, GP * K)]
                handles.append(pltpu.async_copy(t_hbm.at[isl], dst, sem))
            pltpu.sync_copy(t_hbm.at[pl.ds(base + p0, SS)], fi_v)
            for h in handles:
                h.wait()

            def pt_body(p, c2):
                r0 = p * K
                for j in range(HC // 16):
                    sl = pl.ds(j * 16, 16)
                    sl2 = pl.ds(HC + j * 16, 16)
                    fij = fi_v[p, sl]
                    for r in range(K):
                        ev[r0 + r, sl] = rows_v[r0 + r, sl] - fij
                        ev[r0 + r, sl2] = fij
                return c2

            lax.fori_loop(0, SS, pt_body, 0)
            pltpu.sync_copy(ev, e_hbm.at[pl.ds((base + p0) * K, SS * K)])
            return carry

        lax.fori_loop(0, NSUP, sup_body, 0)

    return eg


def _edge_features(table, idx_flat, EW):
    SS = 8 if EW == 256 else 16
    return _edge_gather(EW, SS, 4)(table, idx_flat)


@functools.lru_cache(maxsize=None)
def _edge_mm(EW, O, OPAD):
    blocks = NBN // NP

    def body(e_ref, w_ref, out_ref):
        h = lax.dot_general(e_ref[...], w_ref[...], (((1,), (0,)), ((), ())),
                            preferred_element_type=jnp.float32)
        h = _leaky(h)
        m = jnp.max(h.reshape(NP, K, O), axis=1)
        if OPAD != O:
            m = jnp.concatenate(
                [m, jnp.zeros((NP, OPAD - O), jnp.float32)], axis=1)
        out_ref[...] = m

    return pl.pallas_call(
        body,
        grid=(blocks,),
        in_specs=[
            pl.BlockSpec((NP * K, EW), lambda t: (t, 0)),
            pl.BlockSpec((EW, O), lambda t: (0, 0)),
        ],
        out_specs=pl.BlockSpec((NP, OPAD), lambda t: (t, 0)),
        out_shape=jax.ShapeDtypeStruct((NBN, OPAD), jnp.float32),
    )


@functools.lru_cache(maxsize=None)
def _pool_tc():
    nt_blocks = N // NT2

    def body(f1_ref, f2_ref, f3_ref, f4_ref, w_ref, out_ref):
        t = pl.program_id(1)
        cat = jnp.concatenate(
            [f1_ref[...][:, :64], f2_ref[...][:, :64],
             f3_ref[...], f4_ref[...]], axis=1)
        h = lax.dot_general(cat, w_ref[...], (((1,), (0,)), ((), ())),
                            preferred_element_type=jnp.float32)
        part = jnp.max(_leaky(h), axis=0, keepdims=True)

        @pl.when(t == 0)
        def _init():
            out_ref[0] = part

        @pl.when(t != 0)
        def _acc():
            out_ref[0] = jnp.maximum(out_ref[0], part)

    return pl.pallas_call(
        body,
        grid=(NB, nt_blocks),
        in_specs=[
            pl.BlockSpec((NT2, 128), lambda b, t: (b * nt_blocks + t, 0)),
            pl.BlockSpec((NT2, 128), lambda b, t: (b * nt_blocks + t, 0)),
            pl.BlockSpec((NT2, 128), lambda b, t: (b * nt_blocks + t, 0)),
            pl.BlockSpec((NT2, 256), lambda b, t: (b * nt_blocks + t, 0)),
            pl.BlockSpec((512, 1024), lambda b, t: (0, 0)),
        ],
        out_specs=pl.BlockSpec((1, 1, 1024), lambda b, t: (b, 0, 0)),
        out_shape=jax.ShapeDtypeStruct((NB, 1, 1024), jnp.float32),
    )


@functools.lru_cache(maxsize=None)
def _head_tc():
    def body(nx_ref, y_ref, wy1_ref, by1_ref, wy2_ref, by2_ref,
             l0_ref, l1_ref, l2_ref, bl2_ref, out_ref):
        dn = (((1,), (0,)), ((), ()))
        yy = _leaky(lax.dot_general(y_ref[...], wy1_ref[...], dn,
                                    preferred_element_type=jnp.float32)
                    + by1_ref[...])
        yy = _leaky(lax.dot_general(yy, wy2_ref[...], dn,
                                    preferred_element_type=jnp.float32)
                    + by2_ref[...])
        z = jnp.concatenate([nx_ref[...], yy], axis=1)
        z = _leaky(lax.dot_general(z, l0_ref[...], dn,
                                   preferred_element_type=jnp.float32))
        z = _leaky(lax.dot_general(z, l1_ref[...], dn,
                                   preferred_element_type=jnp.float32))
        out_ref[...] = lax.dot_general(z, l2_ref[...], dn,
                                       preferred_element_type=jnp.float32) \
            + bl2_ref[...]

    return pl.pallas_call(
        body,
        out_shape=jax.ShapeDtypeStruct((B, 1), jnp.float32),
    )


def _chain(xc, wcats, w4t):
    f = xc.reshape(NBN, 6)
    table = jnp.pad(f, ((0, 0), (0, TW - 6)))
    ft = jnp.transpose(xc, (0, 2, 1))

    feats = []
    for (wcat, C, O) in wcats:
        FW = TW if C != 6 else 6
        idx = _knn_tc(C, FW)(f if C == 6 else feats[-1], ft)
        HC = max(16, C)
        EW = 2 * HC
        e = _edge_features(table, idx.reshape(NBN * K), EW)
        OPAD = max(O, TW)
        fpad = _edge_mm(EW, O, OPAD)(e, wcat)
        feats.append(fpad)
        if O != 256:
            table = fpad
            ft = jnp.transpose(fpad[:, :O].reshape(NB, N, O), (0, 2, 1))

    return _pool_tc()(feats[0], feats[1], feats[2], feats[3], w4t)


def kernel(x, y, W0, W1, W2, W3, W4, Wy1, by1, Wy2, by2, L0, L1, L2, bL2):
    wcats = []
    for W, C, O in ((W0, 6, 64), (W1, 64, 64), (W2, 64, 128), (W3, 128, 256)):
        HC = max(16, C)
        wcat = jnp.zeros((2 * HC, O), jnp.float32)
        wcat = wcat.at[:C].set(W[:, :C].T).at[HC:HC + C].set(W[:, C:].T)
        wcats.append((wcat, C, O))
    w4t = W4.T

    nxa = _chain(x[:NB], wcats, w4t)
    nxb = _chain(x[NB:], wcats, w4t)
    nx = jnp.concatenate([nxa, nxb], axis=0).reshape(B, 1024)

    z = _head_tc()(nx, y, Wy1.T, by1.reshape(1, 16), Wy2.T,
                   by2.reshape(1, 64), L0.T, L1.T, L2.T, bL2.reshape(1, 1))
    return z
